# Initial kernel scaffold; baseline (speedup 1.0000x reference)
import jax
import jax.numpy as jnp
from jax.experimental import pallas as pl


def kernel(item_scores, feat_scores, cand_item):
    raise NotImplementedError("write your pallas kernel here")


# speedup vs baseline: 1.6851x; 1.6851x over previous
import functools

import jax
import jax.numpy as jnp
import numpy as np
from jax import lax
from jax.experimental import pallas as pl
from jax.experimental.pallas import tpu as pltpu
from jax.experimental.pallas import tpu_sc as plsc

B = 128
V = 100000
N_FEAT = 25
TOPK = 10
LANES = 16
GROUP = 10
VREGS = V // LANES
GROUPS = VREGS // GROUP
FPAD = 32
OPAD = 48
NEG = np.float32(-1e30)


def _merge_top16(t_asc, x):
    x_asc = lax.sort(x)
    return lax.sort(jnp.maximum(t_asc, lax.rev(x_asc, (0,))))


def _scan_row(row_v, row):
    init = jnp.full((LANES,), np.float32(-np.inf), jnp.float32)

    def group_body(g, carry):
        t_asc, thr = carry
        base = g * jnp.int32(GROUP * LANES)
        xs = [row_v[pl.ds(base + jnp.int32(j * LANES), LANES)]
              for j in range(GROUP)]
        m = xs[0]
        for x in xs[1:]:
            m = jnp.maximum(m, x)
        hit = jnp.max(m) > thr

        def do_merge(c):
            t, _ = c
            for x in xs:
                t = _merge_top16(t, x)
            return t, jnp.min(t)

        return lax.cond(hit, do_merge, lambda c: c, (t_asc, thr))

    t_asc, _ = lax.fori_loop(jnp.int32(0), jnp.int32(GROUPS), group_body,
                             (init, np.float32(-np.inf)))
    return t_asc


def _softmax_row(act_v):
    v0 = act_v[pl.ds(0, LANES)]
    v1 = act_v[pl.ds(16, LANES)]
    v2 = act_v[pl.ds(32, LANES)]
    io = lax.iota(jnp.int32, LANES)
    v1 = jnp.where(io < (N_FEAT - LANES), v1, NEG)
    v2 = jnp.where(io < TOPK, v2, NEG)
    mx = jnp.maximum(jnp.maximum(jnp.max(v0), jnp.max(v1)), jnp.max(v2))
    mxv = jnp.full((LANES,), mx, jnp.float32)
    e0 = jnp.exp(v0 - mxv)
    e1 = jnp.exp(v1 - mxv)
    e2 = jnp.exp(v2 - mxv)
    s = jnp.sum(e0) + jnp.sum(e1) + jnp.sum(e2)
    inv = jnp.full((LANES,), np.float32(1.0), jnp.float32) / jnp.full(
        (LANES,), s, jnp.float32)
    act_v[pl.ds(0, LANES)] = e0 * inv
    act_v[pl.ds(16, LANES)] = e1 * inv
    act_v[pl.ds(32, LANES)] = e2 * inv


def _make_sc_call():
    info = plsc.get_sparse_core_info()
    nw = info.num_cores * info.num_subcores
    rows_per_w = B // nw
    mesh = plsc.VectorSubcoreMesh(core_axis_name="c", subcore_axis_name="s")

    @functools.partial(
        pl.kernel,
        mesh=mesh,
        out_type=jax.ShapeDtypeStruct((B * OPAD,), jnp.float32),
        scratch_types=[
            pltpu.VMEM((V,), jnp.float32),
            pltpu.VMEM((OPAD,), jnp.float32),
        ],
        compiler_params=pltpu.CompilerParams(needs_layout_passes=False),
    )
    def sc_topk(item_hbm, feat_hbm, out_hbm, row_v, act_v):
        wid = (lax.axis_index("s") * jnp.int32(info.num_cores)
               + lax.axis_index("c"))
        base = wid * jnp.int32(rows_per_w)
        for k in range(rows_per_w):
            row = base + jnp.int32(k)
            pltpu.sync_copy(item_hbm.at[pl.ds(row * jnp.int32(V), V)], row_v)
            t_asc = _scan_row(row_v, row)
            pltpu.sync_copy(feat_hbm.at[pl.ds(row * jnp.int32(FPAD), FPAD)],
                            act_v.at[pl.ds(0, FPAD)])
            act_v[pl.ds(32, LANES)] = lax.rev(t_asc, (0,))
            _softmax_row(act_v)
            pltpu.sync_copy(act_v, out_hbm.at[pl.ds(row * jnp.int32(OPAD), OPAD)])

    return sc_topk


def kernel(item_scores, feat_scores, cand_item):
    del cand_item
    feat_pad = jnp.pad(feat_scores.astype(jnp.float32),
                       ((0, 0), (0, FPAD - N_FEAT)))
    out = _make_sc_call()(item_scores.astype(jnp.float32).reshape(-1),
                          feat_pad.reshape(-1))
    r = out.reshape(B, OPAD)
    return jnp.concatenate([r[:, :N_FEAT], r[:, 32:32 + TOPK]], axis=1)
